# baseline (speedup 1.0000x reference)
import functools

import jax
import jax.numpy as jnp
import numpy as np
from jax import lax
from jax.experimental import pallas as pl
from jax.experimental.pallas import tpu as pltpu
from jax.experimental.pallas import tpu_sc as plsc

_BETA = 0.25
_BT = 1280
_NW = 32
_CH = 128


def _dist_argmin_body(z_ref, c_ref, one_ref, idx_ref, loss_ref):
    i = pl.program_id(0)
    zb3 = z_ref[...]
    zb = zb3.reshape(zb3.shape[0] * zb3.shape[1], zb3.shape[2])
    cb = c_ref[...]
    prod_t = lax.dot_general(cb, zb, (((1,), (1,)), ((), ())),
                             preferred_element_type=jnp.float32)
    z_sq = lax.dot_general(one_ref[...], zb * zb,
                           (((1,), (1,)), ((), ())),
                           preferred_element_type=jnp.float32)[:1, :]
    c_sq = jnp.sum(cb * cb, axis=1, keepdims=True)
    dist_t = z_sq - 2.0 * prod_t + c_sq
    m = jnp.min(dist_t, axis=0, keepdims=True)
    k = dist_t.shape[0]
    ks = lax.broadcasted_iota(jnp.int32, dist_t.shape, 0)
    idxb = jnp.min(jnp.where(dist_t == m, ks, k), axis=0)
    idx_ref[0, 0, :] = idxb

    @pl.when(i == 0)
    def _():
        loss_ref[0, 0] = 0.0

    loss_ref[0, 0] += jnp.sum(m)


def _dist_argmin(z, codebook, row_off, rows):
    b, n, d = z.shape
    k = codebook.shape[0]
    br = _BT // n
    bt = br * n
    nb = rows * n // bt
    blk_off = row_off // br
    idx3, loss = pl.pallas_call(
        _dist_argmin_body,
        grid=(nb,),
        in_specs=[
            pl.BlockSpec((br, n, d), lambda i: (i + blk_off, 0, 0)),
            pl.BlockSpec((k, d), lambda i: (0, 0)),
            pl.BlockSpec((8, d), lambda i: (0, 0)),
        ],
        out_specs=[
            pl.BlockSpec((1, 1, bt), lambda i: (i, 0, 0)),
            pl.BlockSpec((1, 1), lambda i: (0, 0), memory_space=pltpu.SMEM),
        ],
        out_shape=[
            jax.ShapeDtypeStruct((nb, 1, bt), jnp.int32),
            jax.ShapeDtypeStruct((1, 1), jnp.float32),
        ],
    )(z, codebook, jnp.ones((8, d), jnp.float32))
    return idx3.reshape(rows * n), loss[0, 0]


def _sc_gather(codebook, idx):
    t = idx.shape[0]
    k, d = codebook.shape
    tok_per_w = t // _NW
    nch = tok_per_w // _CH
    idx3 = idx.reshape(_NW, nch, _CH)

    @functools.partial(
        pl.kernel,
        mesh=plsc.VectorSubcoreMesh(core_axis_name="c", subcore_axis_name="s"),
        out_type=jax.ShapeDtypeStruct((t, d), jnp.float32),
        scratch_types=[
            pltpu.VMEM((nch, _CH), jnp.int32),
            pltpu.VMEM((_CH, d), jnp.float32),
            pltpu.VMEM((_CH, d), jnp.float32),
            pltpu.VMEM((_CH, d), jnp.float32),
            pltpu.SemaphoreType.DMA,
            pltpu.SemaphoreType.DMA,
            pltpu.SemaphoreType.DMA,
            pltpu.SemaphoreType.DMA,
            pltpu.SemaphoreType.DMA,
            pltpu.SemaphoreType.DMA,
        ],
    )
    def gather_kernel(cb_hbm, idx_hbm, out_hbm, idx_v,
                      buf0, buf1, buf2, g0, g1, g2, s0, s1, s2):
        cid = lax.axis_index("c")
        sid = lax.axis_index("s")
        wid = sid * 2 + cid
        base = pl.multiple_of(wid * tok_per_w, _CH)
        pltpu.sync_copy(idx_hbm.at[wid], idx_v)
        bufs = (buf0, buf1, buf2)
        gsems = (g0, g1, g2)
        ssems = (s0, s1, s2)
        gth = [None, None, None]
        st = [None, None, None]
        for j in range(min(3, nch)):
            gth[j] = pltpu.async_copy(cb_hbm.at[idx_v.at[j]], bufs[j], gsems[j])
        for j in range(nch):
            p = j % 3
            gth[p].wait()
            st[p] = pltpu.async_copy(
                bufs[p], out_hbm.at[pl.ds(base + j * _CH, _CH)], ssems[p])
            if j + 3 < nch:
                st[p].wait()
                gth[p] = pltpu.async_copy(
                    cb_hbm.at[idx_v.at[j + 3]], bufs[p], gsems[p])
        for j in range(max(0, nch - 3), nch):
            st[j % 3].wait()

    return gather_kernel(codebook, idx3)


def kernel(z, codebook):
    b, n, d = z.shape
    t = b * n
    half = b // 2
    idx1, loss1 = _dist_argmin(z, codebook, 0, half)
    idx2, loss2 = _dist_argmin(z, codebook, half, b - half)
    zq1 = _sc_gather(codebook, idx1)
    zq2 = _sc_gather(codebook, idx2)
    z_q = jnp.concatenate([zq1, zq2], axis=0)
    idx_flat = jnp.concatenate([idx1, idx2], axis=0)
    vq_loss = (_BETA / (t * d)) * (loss1 + loss2)
    return (z_q.reshape(b, n, d), vq_loss, idx_flat.reshape(b, n))


# speedup vs baseline: 1.0345x; 1.0345x over previous
import functools

import jax
import jax.numpy as jnp
import numpy as np
from jax import lax
from jax.experimental import pallas as pl
from jax.experimental.pallas import tpu as pltpu
from jax.experimental.pallas import tpu_sc as plsc

_BETA = 0.25
_BT = 2560
_NW = 32
_CH = 128


def _dist_argmin_body(z_ref, c_ref, one_ref, idx_ref, loss_ref):
    i = pl.program_id(0)
    zb3 = z_ref[...]
    zb = zb3.reshape(zb3.shape[0] * zb3.shape[1], zb3.shape[2])
    cb = c_ref[...]
    prod_t = lax.dot_general(cb, zb, (((1,), (1,)), ((), ())),
                             preferred_element_type=jnp.float32)
    z_sq = lax.dot_general(one_ref[...], zb * zb,
                           (((1,), (1,)), ((), ())),
                           preferred_element_type=jnp.float32)[:1, :]
    c_sq = jnp.sum(cb * cb, axis=1, keepdims=True)
    dist_t = z_sq - 2.0 * prod_t + c_sq
    m = jnp.min(dist_t, axis=0, keepdims=True)
    k = dist_t.shape[0]
    ks = lax.broadcasted_iota(jnp.int32, dist_t.shape, 0)
    idxb = jnp.min(jnp.where(dist_t == m, ks, k), axis=0)
    idx_ref[0, 0, :] = idxb

    @pl.when(i == 0)
    def _():
        loss_ref[0, 0] = 0.0

    loss_ref[0, 0] += jnp.sum(m)


def _dist_argmin(z, codebook, row_off, rows):
    b, n, d = z.shape
    k = codebook.shape[0]
    br = _BT // n
    bt = br * n
    nb = rows * n // bt
    blk_off = row_off // br
    idx3, loss = pl.pallas_call(
        _dist_argmin_body,
        grid=(nb,),
        in_specs=[
            pl.BlockSpec((br, n, d), lambda i: (i + blk_off, 0, 0)),
            pl.BlockSpec((k, d), lambda i: (0, 0)),
            pl.BlockSpec((8, d), lambda i: (0, 0)),
        ],
        out_specs=[
            pl.BlockSpec((1, 1, bt), lambda i: (i, 0, 0)),
            pl.BlockSpec((1, 1), lambda i: (0, 0), memory_space=pltpu.SMEM),
        ],
        out_shape=[
            jax.ShapeDtypeStruct((nb, 1, bt), jnp.int32),
            jax.ShapeDtypeStruct((1, 1), jnp.float32),
        ],
    )(z, codebook, jnp.ones((8, d), jnp.float32))
    return idx3.reshape(rows * n), loss[0, 0]


def _sc_gather(codebook, idx):
    t = idx.shape[0]
    k, d = codebook.shape
    tok_per_w = t // _NW
    nch = tok_per_w // _CH
    idx3 = idx.reshape(_NW, nch, _CH)

    @functools.partial(
        pl.kernel,
        mesh=plsc.VectorSubcoreMesh(core_axis_name="c", subcore_axis_name="s"),
        out_type=jax.ShapeDtypeStruct((t, d), jnp.float32),
        scratch_types=[
            pltpu.VMEM((nch, _CH), jnp.int32),
            pltpu.VMEM((_CH, d), jnp.float32),
            pltpu.VMEM((_CH, d), jnp.float32),
            pltpu.VMEM((_CH, d), jnp.float32),
            pltpu.SemaphoreType.DMA,
            pltpu.SemaphoreType.DMA,
            pltpu.SemaphoreType.DMA,
            pltpu.SemaphoreType.DMA,
            pltpu.SemaphoreType.DMA,
            pltpu.SemaphoreType.DMA,
        ],
    )
    def gather_kernel(cb_hbm, idx_hbm, out_hbm, idx_v,
                      buf0, buf1, buf2, g0, g1, g2, s0, s1, s2):
        cid = lax.axis_index("c")
        sid = lax.axis_index("s")
        wid = sid * 2 + cid
        base = pl.multiple_of(wid * tok_per_w, _CH)
        pltpu.sync_copy(idx_hbm.at[wid], idx_v)
        bufs = (buf0, buf1, buf2)
        gsems = (g0, g1, g2)
        ssems = (s0, s1, s2)
        gth = [None, None, None]
        st = [None, None, None]
        for j in range(min(3, nch)):
            gth[j] = pltpu.async_copy(cb_hbm.at[idx_v.at[j]], bufs[j], gsems[j])
        for j in range(nch):
            p = j % 3
            gth[p].wait()
            st[p] = pltpu.async_copy(
                bufs[p], out_hbm.at[pl.ds(base + j * _CH, _CH)], ssems[p])
            if j + 3 < nch:
                st[p].wait()
                gth[p] = pltpu.async_copy(
                    cb_hbm.at[idx_v.at[j + 3]], bufs[p], gsems[p])
        for j in range(max(0, nch - 3), nch):
            st[j % 3].wait()

    return gather_kernel(codebook, idx3)


def kernel(z, codebook):
    b, n, d = z.shape
    t = b * n
    idx_flat, loss_sum = _dist_argmin(z, codebook, 0, b)
    z_q = _sc_gather(codebook, idx_flat)
    vq_loss = (_BETA / (t * d)) * loss_sum
    return (z_q.reshape(b, n, d), vq_loss, idx_flat.reshape(b, n))


# speedup vs baseline: 1.0957x; 1.0592x over previous
import functools

import jax
import jax.numpy as jnp
import numpy as np
from jax import lax
from jax.experimental import pallas as pl
from jax.experimental.pallas import tpu as pltpu
from jax.experimental.pallas import tpu_sc as plsc

_BETA = 0.25
_BT = 2560
_NW = 32
_CH = 128


def _dist_argmin_body(z_ref, c_ref, one_ref, idx_ref, loss_ref):
    i = pl.program_id(0)
    zb3 = z_ref[...]
    zb = zb3.reshape(zb3.shape[0] * zb3.shape[1], zb3.shape[2])
    cb = c_ref[...]
    prod_t = lax.dot_general(cb, zb, (((1,), (1,)), ((), ())),
                             preferred_element_type=jnp.float32)
    zbf = zb.astype(jnp.bfloat16)
    z_sq = lax.dot_general(one_ref[...], zbf * zbf,
                           (((1,), (1,)), ((), ())),
                           preferred_element_type=jnp.float32)
    c_sq = jnp.sum(cb * cb, axis=1, keepdims=True)
    dist_t = z_sq - 2.0 * prod_t + c_sq
    m = jnp.min(dist_t, axis=0, keepdims=True)
    k = dist_t.shape[0]
    ks = lax.broadcasted_iota(jnp.int32, dist_t.shape, 0)
    idxb = jnp.min(jnp.where(dist_t == m, ks, k), axis=0)
    idx_ref[0, 0, :] = idxb

    @pl.when(i == 0)
    def _():
        loss_ref[0, 0] = 0.0

    loss_ref[0, 0] += jnp.sum(m)


def _dist_argmin(z, codebook, row_off, rows):
    b, n, d = z.shape
    k = codebook.shape[0]
    br = _BT // n
    bt = br * n
    nb = rows // br
    blk_off = row_off // br
    idx3, loss = pl.pallas_call(
        _dist_argmin_body,
        grid=(nb,),
        in_specs=[
            pl.BlockSpec((br, n, d), lambda i: (i + blk_off, 0, 0)),
            pl.BlockSpec((k, d), lambda i: (0, 0)),
            pl.BlockSpec((k, d), lambda i: (0, 0)),
        ],
        out_specs=[
            pl.BlockSpec((1, 1, bt), lambda i: (i, 0, 0)),
            pl.BlockSpec((1, 1), lambda i: (0, 0), memory_space=pltpu.SMEM),
        ],
        out_shape=[
            jax.ShapeDtypeStruct((nb, 1, bt), jnp.int32),
            jax.ShapeDtypeStruct((1, 1), jnp.float32),
        ],
    )(z, codebook, jnp.ones((codebook.shape[0], d), jnp.bfloat16))
    return idx3.reshape(rows * n), loss[0, 0]


def _sc_gather(codebook, idx):
    t = idx.shape[0]
    k, d = codebook.shape
    tok_per_w = t // _NW
    nch = tok_per_w // _CH
    idx3 = idx.reshape(_NW, nch, _CH)

    @functools.partial(
        pl.kernel,
        mesh=plsc.VectorSubcoreMesh(core_axis_name="c", subcore_axis_name="s"),
        out_type=jax.ShapeDtypeStruct((t, d), jnp.float32),
        scratch_types=[
            pltpu.VMEM((nch, _CH), jnp.int32),
            pltpu.VMEM((_CH, d), jnp.float32),
            pltpu.VMEM((_CH, d), jnp.float32),
            pltpu.VMEM((_CH, d), jnp.float32),
            pltpu.SemaphoreType.DMA,
            pltpu.SemaphoreType.DMA,
            pltpu.SemaphoreType.DMA,
            pltpu.SemaphoreType.DMA,
            pltpu.SemaphoreType.DMA,
            pltpu.SemaphoreType.DMA,
        ],
    )
    def gather_kernel(cb_hbm, idx_hbm, out_hbm, idx_v,
                      buf0, buf1, buf2, g0, g1, g2, s0, s1, s2):
        cid = lax.axis_index("c")
        sid = lax.axis_index("s")
        wid = sid * 2 + cid
        base = pl.multiple_of(wid * tok_per_w, _CH)

        pltpu.sync_copy(idx_hbm.at[wid], idx_v)
        bufs = (buf0, buf1, buf2)
        gsems = (g0, g1, g2)
        ssems = (s0, s1, s2)
        gth = [None, None, None]
        st = [None, None, None]
        for j in range(min(3, nch)):
            gth[j] = pltpu.async_copy(cb_hbm.at[idx_v.at[j]], bufs[j], gsems[j])
        for j in range(nch):
            p = j % 3
            gth[p].wait()
            st[p] = pltpu.async_copy(
                bufs[p], out_hbm.at[pl.ds(base + j * _CH, _CH)], ssems[p])
            if j + 3 < nch:
                st[p].wait()
                gth[p] = pltpu.async_copy(
                    cb_hbm.at[idx_v.at[j + 3]], bufs[p], gsems[p])
        for j in range(max(0, nch - 3), nch):
            st[j % 3].wait()

    return gather_kernel(codebook, idx3)


def kernel(z, codebook):
    b, n, d = z.shape
    t = b * n
    idx_flat, loss_sum = _dist_argmin(z, codebook, 0, b)
    z_q = _sc_gather(codebook, idx_flat)
    vq_loss = (_BETA / (t * d)) * loss_sum
    return (z_q.reshape(b, n, d), vq_loss, idx_flat.reshape(b, n))


# speedup vs baseline: 1.1259x; 1.0276x over previous
import functools

import jax
import jax.numpy as jnp
import numpy as np
from jax import lax
from jax.experimental import pallas as pl
from jax.experimental.pallas import tpu as pltpu
from jax.experimental.pallas import tpu_sc as plsc

_BETA = 0.25
_BT = 2560
_NW = 32
_CH = 128


def _dist_argmin_body(z_ref, c_ref, one_ref, idx_ref, loss_ref):
    i = pl.program_id(0)
    zb3 = z_ref[...]
    zb = zb3.reshape(zb3.shape[0] * zb3.shape[1], zb3.shape[2])
    cb = c_ref[...]
    prod_t = lax.dot_general(cb * -2.0, zb, (((1,), (1,)), ((), ())),
                             preferred_element_type=jnp.float32)
    zbf = zb.astype(jnp.bfloat16)
    z_sq = lax.dot_general(one_ref[...], zbf * zbf,
                           (((1,), (1,)), ((), ())),
                           preferred_element_type=jnp.float32)
    c_sq = jnp.sum(cb * cb, axis=1, keepdims=True)
    dist_t = (z_sq + prod_t) + c_sq
    m = jnp.min(dist_t, axis=0, keepdims=True)
    k = dist_t.shape[0]
    ks = lax.broadcasted_iota(jnp.int32, dist_t.shape, 0)
    idxb = jnp.min(jnp.where(dist_t == m, ks, k), axis=0)
    nch = idx_ref.shape[1] - 4
    idx_ref[0, :nch, :] = idxb.reshape(nch, 128)
    idx_ref[0, nch:, :] = jnp.zeros((4, 128), jnp.int32)

    @pl.when(i == 0)
    def _():
        loss_ref[0, 0] = 0.0

    loss_ref[0, 0] += jnp.sum(m)


def _dist_argmin(z, codebook, row_off, rows):
    b, n, d = z.shape
    k = codebook.shape[0]
    br = _BT // n
    bt = br * n
    nb = rows // br
    blk_off = row_off // br
    idx3, loss = pl.pallas_call(
        _dist_argmin_body,
        grid=(nb,),
        in_specs=[
            pl.BlockSpec((br, n, d), lambda i: (i + blk_off, 0, 0)),
            pl.BlockSpec((k, d), lambda i: (0, 0)),
            pl.BlockSpec((k, d), lambda i: (0, 0)),
        ],
        out_specs=[
            pl.BlockSpec((1, bt // 128 + 4, 128), lambda i: (i, 0, 0)),
            pl.BlockSpec((1, 1), lambda i: (0, 0), memory_space=pltpu.SMEM),
        ],
        out_shape=[
            jax.ShapeDtypeStruct((nb, bt // 128 + 4, 128), jnp.int32),
            jax.ShapeDtypeStruct((1, 1), jnp.float32),
        ],
    )(z, codebook, jnp.ones((codebook.shape[0], d), jnp.bfloat16))
    return idx3, loss[0, 0]


def _sc_gather(codebook, idx3):
    nch = idx3.shape[1] - 4
    t = _NW * nch * _CH
    k, d = codebook.shape
    tok_per_w = t // _NW

    @functools.partial(
        pl.kernel,
        mesh=plsc.VectorSubcoreMesh(core_axis_name="c", subcore_axis_name="s"),
        out_type=jax.ShapeDtypeStruct((t, d), jnp.float32),
        scratch_types=[
            pltpu.VMEM((nch + 4, _CH), jnp.int32),
            pltpu.VMEM((_CH, d), jnp.float32),
            pltpu.VMEM((_CH, d), jnp.float32),
            pltpu.VMEM((_CH, d), jnp.float32),
            pltpu.SemaphoreType.DMA,
            pltpu.SemaphoreType.DMA,
            pltpu.SemaphoreType.DMA,
            pltpu.SemaphoreType.DMA,
            pltpu.SemaphoreType.DMA,
            pltpu.SemaphoreType.DMA,
        ],
    )
    def gather_kernel(cb_hbm, idx_hbm, out_hbm, idx_v,
                      buf0, buf1, buf2, g0, g1, g2, s0, s1, s2):
        cid = lax.axis_index("c")
        sid = lax.axis_index("s")
        wid = sid * 2 + cid
        base = pl.multiple_of(wid * tok_per_w, _CH)

        pltpu.sync_copy(idx_hbm.at[wid], idx_v)
        bufs = (buf0, buf1, buf2)
        gsems = (g0, g1, g2)
        ssems = (s0, s1, s2)
        gth = [None, None, None]
        st = [None, None, None]
        for j in range(min(3, nch)):
            gth[j] = pltpu.async_copy(cb_hbm.at[idx_v.at[j]], bufs[j], gsems[j])
        for j in range(nch):
            p = j % 3
            gth[p].wait()
            st[p] = pltpu.async_copy(
                bufs[p], out_hbm.at[pl.ds(base + j * _CH, _CH)], ssems[p])
            if j + 3 < nch:
                st[p].wait()
                gth[p] = pltpu.async_copy(
                    cb_hbm.at[idx_v.at[j + 3]], bufs[p], gsems[p])
        for j in range(max(0, nch - 3), nch):
            st[j % 3].wait()

    return gather_kernel(codebook, idx3)


def kernel(z, codebook):
    b, n, d = z.shape
    t = b * n
    idx24, loss_sum = _dist_argmin(z, codebook, 0, b)
    z_q = _sc_gather(codebook, idx24)
    idx_flat = idx24[:, :idx24.shape[1] - 4, :].reshape(t)
    vq_loss = (_BETA / (t * d)) * loss_sum
    return (z_q.reshape(b, n, d), vq_loss, idx_flat.reshape(b, n))
